# baseline (speedup 1.0000x reference)
import functools

import numpy as np

import jax
import jax.numpy as jnp
from jax import lax
from jax.experimental import pallas as pl
from jax.experimental.pallas import tpu as pltpu
from jax.experimental.pallas import tpu_sc as plsc

N_NODES = 100000
N_EDGES = 1600000
D_NODE = 128
D_EDGE = 16
D_TIME = 128
D_EMB = 128
B = 4096
NBR = 20

NW = 32
SRC_PER_W = B // NW
CHUNK_SRC = 4
CHUNK_ROWS = CHUNK_SRC * NBR
N_CHUNKS = SRC_PER_W // CHUNK_SRC

EGROUPS = N_EDGES // 8
ECH_SRC = 16
ECH_ROWS = ECH_SRC * NBR
ECH_N = SRC_PER_W // ECH_SRC


RPK_CW = 512
RPK_NC = N_EDGES // RPK_CW


def _sc_repack_body(tabT_hbm, out_hbm, bufa, bufo, sem):
    wid = lax.axis_index("s") * 2 + lax.axis_index("c")
    n_w = (RPK_NC // NW) + (wid < (RPK_NC % NW)).astype(jnp.int32)
    lanes = lax.iota(jnp.int32, 16)
    z16 = lanes * 0

    def chunk_body(i, carry):
        cid = wid + NW * i
        ioff = pl.multiple_of(cid * RPK_CW, 128)
        pltpu.sync_copy(tabT_hbm.at[:, pl.ds(ioff, RPK_CW)],
                        bufa.at[:, pl.ds(0, RPK_CW)])
        for x in range(RPK_CW):
            v = plsc.load_gather(bufa, [lanes, z16 + x])
            bufo[x // 8, (x % 8) * D_EDGE:(x % 8) * D_EDGE + D_EDGE] = v
        ooff = pl.multiple_of(cid * (RPK_CW // 8), 8)
        pltpu.sync_copy(bufo, out_hbm.at[pl.ds(ooff, RPK_CW // 8)])
        return carry

    lax.fori_loop(0, n_w, chunk_body, 0)


@functools.lru_cache(maxsize=1)
def _build_sc_repack():
    return functools.partial(
        pl.kernel,
        out_type=(jax.ShapeDtypeStruct((EGROUPS, 128), jnp.float32),),
        mesh=plsc.VectorSubcoreMesh(core_axis_name="c", subcore_axis_name="s",
                                    num_cores=2, num_subcores=16),
        compiler_params=pltpu.CompilerParams(needs_layout_passes=False),
        scratch_types=(
            pltpu.VMEM((D_EDGE, RPK_CW + 1), jnp.float32),
            pltpu.VMEM((RPK_CW // 8, 128), jnp.float32),
            pltpu.SemaphoreType.DMA,
        ),
    )(_sc_repack_body)


def _repack_edges(edge_features):
    out = _build_sc_repack()(edge_features.T)
    if isinstance(out, (tuple, list)):
        out = out[0]
    return out


def _sc_nodes_body(node_hbm, mem_hbm, nbr_hbm, src_hbm,
                   out_nb, out_sn, out_sm,
                   idx_nb, idx_s, buf_n, buf_m, acc_nb, bsn, bsm, sem):
    wid = lax.axis_index("s") * 2 + lax.axis_index("c")
    row0 = wid * (SRC_PER_W * NBR)
    src0 = wid * SRC_PER_W

    pltpu.sync_copy(nbr_hbm.at[pl.ds(row0, SRC_PER_W * NBR)], idx_nb)
    pltpu.sync_copy(src_hbm.at[pl.ds(src0, SRC_PER_W)], idx_s)

    def chunk_body(c, carry):
        r = c * CHUNK_ROWS
        cp_n = pltpu.async_copy(node_hbm.at[idx_nb.at[pl.ds(r, CHUNK_ROWS)]],
                                buf_n, sem)
        cp_m = pltpu.async_copy(mem_hbm.at[idx_nb.at[pl.ds(r, CHUNK_ROWS)]],
                                buf_m, sem)
        cp_n.wait()
        cp_m.wait()
        for s in range(CHUNK_SRC):
            base = s * NBR
            orow = c * CHUNK_SRC + s
            for col in range(D_NODE // 16):
                sl = pl.ds(col * 16, 16)
                acc = buf_n[base, sl] + buf_m[base, sl]
                for j in range(1, NBR):
                    acc = acc + buf_n[base + j, sl] + buf_m[base + j, sl]
                acc_nb[orow, sl] = acc
        return carry

    lax.fori_loop(0, N_CHUNKS, chunk_body, 0)

    cp_sn = pltpu.async_copy(node_hbm.at[idx_s], bsn, sem)
    cp_sm = pltpu.async_copy(mem_hbm.at[idx_s], bsm, sem)
    cp_sn.wait()
    cp_sm.wait()

    pltpu.sync_copy(acc_nb, out_nb.at[pl.ds(src0, SRC_PER_W)])
    pltpu.sync_copy(bsn, out_sn.at[pl.ds(src0, SRC_PER_W)])
    pltpu.sync_copy(bsm, out_sm.at[pl.ds(src0, SRC_PER_W)])


@functools.lru_cache(maxsize=1)
def _build_sc_nodes():
    return functools.partial(
        pl.kernel,
        out_type=(
            jax.ShapeDtypeStruct((B, D_NODE), jnp.float32),
            jax.ShapeDtypeStruct((B, D_NODE), jnp.float32),
            jax.ShapeDtypeStruct((B, D_NODE), jnp.float32),
        ),
        mesh=plsc.VectorSubcoreMesh(core_axis_name="c", subcore_axis_name="s",
                                    num_cores=2, num_subcores=16),
        scratch_types=(
            pltpu.VMEM((SRC_PER_W * NBR,), jnp.int32),
            pltpu.VMEM((SRC_PER_W,), jnp.int32),
            pltpu.VMEM((CHUNK_ROWS, D_NODE), jnp.float32),
            pltpu.VMEM((CHUNK_ROWS, D_NODE), jnp.float32),
            pltpu.VMEM((SRC_PER_W, D_NODE), jnp.float32),
            pltpu.VMEM((SRC_PER_W, D_NODE), jnp.float32),
            pltpu.VMEM((SRC_PER_W, D_NODE), jnp.float32),
            pltpu.SemaphoreType.DMA,
        ),
    )(_sc_nodes_body)


def _sc_edges_body(dense_hbm, eidx_hbm, out3, idx_e, gidx, gbuf, acc_t, sem):
    wid = lax.axis_index("s") * 2 + lax.axis_index("c")
    row0 = wid * (SRC_PER_W * NBR)

    pltpu.sync_copy(eidx_hbm.at[pl.ds(row0, SRC_PER_W * NBR)], idx_e)

    lanes = lax.iota(jnp.int32, 16)

    lanes20 = lanes * NBR
    zero16 = jnp.float32(0.0) + jnp.zeros_like(lanes, dtype=jnp.float32)

    def chunk_body(c, carry):
        bc = jax.lax.broadcast_in_dim(c, (16,), ()) * ECH_ROWS
        for t in range(ECH_ROWS // 16):
            cslots = lanes + 16 * t
            ev = plsc.load_gather(idx_e, [bc + cslots])
            plsc.store_scatter(gidx, [cslots], ev >> 3)
        cps = []
        for q in range(ECH_ROWS // 80):
            cps.append(pltpu.async_copy(
                dense_hbm.at[gidx.at[pl.ds(q * 80, 80)]],
                gbuf.at[pl.ds(q * 80, 80)], sem))
        for cp in cps:
            cp.wait()
        bc16 = jax.lax.broadcast_in_dim(c, (16,), ()) * ECH_SRC
        accs = [zero16 for _ in range(D_EDGE)]
        for j in range(NBR):
            rows = lanes20 + j
            sub = plsc.load_gather(idx_e, [bc + rows])
            colb = (sub & 7) * 16
            for k in range(D_EDGE):
                vals = plsc.load_gather(gbuf, [rows, colb + k])
                accs[k] = accs[k] + vals
        for k in range(D_EDGE):
            krow = jax.lax.broadcast_in_dim(jnp.int32(k), (16,), ())
            plsc.store_scatter(acc_t, [krow, bc16 + lanes], accs[k])
        return carry

    lax.fori_loop(0, ECH_N, chunk_body, 0)
    pltpu.sync_copy(acc_t, out3.at[wid])


@functools.lru_cache(maxsize=1)
def _build_sc_edges():
    return functools.partial(
        pl.kernel,
        out_type=(
            jax.ShapeDtypeStruct((NW, D_EDGE, SRC_PER_W), jnp.float32),
        ),
        mesh=plsc.VectorSubcoreMesh(core_axis_name="c", subcore_axis_name="s",
                                    num_cores=2, num_subcores=16),
        compiler_params=pltpu.CompilerParams(needs_layout_passes=False),
        scratch_types=(
            pltpu.VMEM((SRC_PER_W * NBR,), jnp.int32),
            pltpu.VMEM((ECH_ROWS,), jnp.int32),
            pltpu.VMEM((ECH_ROWS, 128), jnp.float32),
            pltpu.VMEM((D_EDGE, SRC_PER_W), jnp.float32),
            pltpu.SemaphoreType.DMA,
        ),
    )(_sc_edges_body)


def _tc_body(et_ref, nb_ref, e3_ref, sn_ref, sm_ref, w_ref, tb_ref,
             w1n_ref, w1t_ref, w1e_ref, w2n_ref, w2s_ref, w2t_ref,
             b1_ref, b2_ref, out_ref):
    ts = et_ref[:, NBR:NBR + 1]
    w = w_ref[...]
    tb = tb_ref[...]
    tsum = jnp.cos((ts - et_ref[:, 0:1]) * w + tb)
    for j in range(1, NBR):
        tsum = tsum + jnp.cos((ts - et_ref[:, j:j + 1]) * w + tb)

    dot = functools.partial(jnp.dot, preferred_element_type=jnp.float32)
    cdims = (((0,), (0,)), ((), ()))
    econ = jnp.concatenate(
        [lax.dot_general(e3_ref[u], w1e_ref[...], cdims,
                         preferred_element_type=jnp.float32)
         for u in range(2)], axis=0)
    pre = (dot(nb_ref[...], w1n_ref[...]) + dot(tsum, w1t_ref[...])
           + econ + float(NBR) * b1_ref[...])
    ns = jnp.maximum(pre, 0.0)
    src = sn_ref[...] + sm_ref[...]
    const = dot(jnp.cos(tb), w2t_ref[...]) + b2_ref[...]
    out_ref[...] = dot(ns, w2n_ref[...]) + dot(src, w2s_ref[...]) + const


def kernel(source_nodes, timestamps, neighbors, edge_idxs, edge_times,
           node_features, edge_features, memory, time_w, time_b,
           W1, b1, W2, b2):
    nbr_flat = neighbors.reshape(-1).astype(jnp.int32)
    eidx_flat = edge_idxs.reshape(-1).astype(jnp.int32)
    src = source_nodes.astype(jnp.int32)

    edge_dense = _repack_edges(edge_features)

    nb_sum, src_n, src_m = _build_sc_nodes()(
        node_features, memory, nbr_flat, src)

    e3 = _build_sc_edges()(edge_dense, eidx_flat)
    if isinstance(e3, (tuple, list)):
        e3 = e3[0]

    et_aug = jnp.concatenate([edge_times, timestamps[:, None]], axis=1)

    blk = 256
    grid = B // blk
    full = lambda i: (0, 0)
    row = lambda i: (i, 0)
    out = pl.pallas_call(
        _tc_body,
        grid=(grid,),
        in_specs=[
            pl.BlockSpec((blk, NBR + 1), row),
            pl.BlockSpec((blk, D_NODE), row),
            pl.BlockSpec((2, D_EDGE, SRC_PER_W), lambda i: (i, 0, 0)),
            pl.BlockSpec((blk, D_NODE), row),
            pl.BlockSpec((blk, D_NODE), row),
            pl.BlockSpec((1, D_TIME), full),
            pl.BlockSpec((1, D_TIME), full),
            pl.BlockSpec((D_NODE, D_EMB), full),
            pl.BlockSpec((D_TIME, D_EMB), full),
            pl.BlockSpec((D_EDGE, D_EMB), full),
            pl.BlockSpec((D_EMB, D_EMB), full),
            pl.BlockSpec((D_NODE, D_EMB), full),
            pl.BlockSpec((D_TIME, D_EMB), full),
            pl.BlockSpec((1, D_EMB), full),
            pl.BlockSpec((1, D_EMB), full),
        ],
        out_specs=pl.BlockSpec((blk, D_EMB), row),
        out_shape=jax.ShapeDtypeStruct((B, D_EMB), jnp.float32),
    )(
        et_aug, nb_sum, e3, src_n, src_m,
        time_w[None, :], time_b[None, :],
        W1[:D_NODE], W1[D_NODE:D_NODE + D_TIME], W1[D_NODE + D_TIME:],
        W2[:D_EMB], W2[D_EMB:D_EMB + D_NODE], W2[D_EMB + D_NODE:],
        b1[None, :], b2[None, :],
    )
    return out


# speedup vs baseline: 1.4535x; 1.4535x over previous
import functools

import numpy as np

import jax
import jax.numpy as jnp
from jax import lax
from jax.experimental import pallas as pl
from jax.experimental.pallas import tpu as pltpu
from jax.experimental.pallas import tpu_sc as plsc

N_NODES = 100000
N_EDGES = 1600000
D_NODE = 128
D_EDGE = 16
D_TIME = 128
D_EMB = 128
B = 4096
NBR = 20

NW = 32
SRC_PER_W = B // NW
CHUNK_SRC = 4
CHUNK_ROWS = CHUNK_SRC * NBR
N_CHUNKS = SRC_PER_W // CHUNK_SRC

EGROUPS = N_EDGES // 8
ECH_SRC = 16
ECH_ROWS = ECH_SRC * NBR
ECH_N = SRC_PER_W // ECH_SRC


RPK_CW = 512
RPK_NC = N_EDGES // RPK_CW


def _sc_repack_body(tabT_hbm, out_hbm, bufa, bufo, sem):
    wid = lax.axis_index("s") * 2 + lax.axis_index("c")
    n_w = (RPK_NC // NW) + (wid < (RPK_NC % NW)).astype(jnp.int32)
    lanes = lax.iota(jnp.int32, 16)
    z16 = lanes * 0

    def chunk_body(i, carry):
        cid = wid + NW * i
        ioff = pl.multiple_of(cid * RPK_CW, 128)
        pltpu.sync_copy(tabT_hbm.at[:, pl.ds(ioff, RPK_CW)],
                        bufa.at[:, pl.ds(0, RPK_CW)])

        @plsc.parallel_loop(0, RPK_CW, unroll=8)
        def _x(x):
            xb = jax.lax.broadcast_in_dim(x, (16,), ())
            v = plsc.load_gather(bufa, [lanes, z16 + x])
            plsc.store_scatter(
                bufo, [xb >> 3, (xb & 7) * D_EDGE + lanes], v)
        ooff = pl.multiple_of(cid * (RPK_CW // 8), 8)
        pltpu.sync_copy(bufo, out_hbm.at[pl.ds(ooff, RPK_CW // 8)])
        return carry

    lax.fori_loop(0, n_w, chunk_body, 0)


@functools.lru_cache(maxsize=1)
def _build_sc_repack():
    return functools.partial(
        pl.kernel,
        out_type=(jax.ShapeDtypeStruct((EGROUPS, 128), jnp.float32),),
        mesh=plsc.VectorSubcoreMesh(core_axis_name="c", subcore_axis_name="s",
                                    num_cores=2, num_subcores=16),
        compiler_params=pltpu.CompilerParams(needs_layout_passes=False),
        scratch_types=(
            pltpu.VMEM((D_EDGE, RPK_CW + 1), jnp.float32),
            pltpu.VMEM((RPK_CW // 8, 128), jnp.float32),
            pltpu.SemaphoreType.DMA,
        ),
    )(_sc_repack_body)


def _repack_edges(edge_features):
    out = _build_sc_repack()(edge_features.T)
    if isinstance(out, (tuple, list)):
        out = out[0]
    return out


def _sc_nodes_body(node_hbm, mem_hbm, nbr_hbm, src_hbm,
                   out_nb, out_sn, out_sm,
                   idx_nb, idx_s, buf_n, buf_m, acc_nb, bsn, bsm, sem):
    wid = lax.axis_index("s") * 2 + lax.axis_index("c")
    row0 = wid * (SRC_PER_W * NBR)
    src0 = wid * SRC_PER_W

    pltpu.sync_copy(nbr_hbm.at[pl.ds(row0, SRC_PER_W * NBR)], idx_nb)
    pltpu.sync_copy(src_hbm.at[pl.ds(src0, SRC_PER_W)], idx_s)

    def chunk_body(c, carry):
        r = c * CHUNK_ROWS
        cp_n = pltpu.async_copy(node_hbm.at[idx_nb.at[pl.ds(r, CHUNK_ROWS)]],
                                buf_n, sem)
        cp_m = pltpu.async_copy(mem_hbm.at[idx_nb.at[pl.ds(r, CHUNK_ROWS)]],
                                buf_m, sem)
        cp_n.wait()
        cp_m.wait()
        for s in range(CHUNK_SRC):
            base = s * NBR
            orow = c * CHUNK_SRC + s
            for col in range(D_NODE // 16):
                sl = pl.ds(col * 16, 16)
                acc = buf_n[base, sl] + buf_m[base, sl]
                for j in range(1, NBR):
                    acc = acc + buf_n[base + j, sl] + buf_m[base + j, sl]
                acc_nb[orow, sl] = acc
        return carry

    lax.fori_loop(0, N_CHUNKS, chunk_body, 0)

    cp_sn = pltpu.async_copy(node_hbm.at[idx_s], bsn, sem)
    cp_sm = pltpu.async_copy(mem_hbm.at[idx_s], bsm, sem)
    cp_sn.wait()
    cp_sm.wait()

    pltpu.sync_copy(acc_nb, out_nb.at[pl.ds(src0, SRC_PER_W)])
    pltpu.sync_copy(bsn, out_sn.at[pl.ds(src0, SRC_PER_W)])
    pltpu.sync_copy(bsm, out_sm.at[pl.ds(src0, SRC_PER_W)])


@functools.lru_cache(maxsize=1)
def _build_sc_nodes():
    return functools.partial(
        pl.kernel,
        out_type=(
            jax.ShapeDtypeStruct((B, D_NODE), jnp.float32),
            jax.ShapeDtypeStruct((B, D_NODE), jnp.float32),
            jax.ShapeDtypeStruct((B, D_NODE), jnp.float32),
        ),
        mesh=plsc.VectorSubcoreMesh(core_axis_name="c", subcore_axis_name="s",
                                    num_cores=2, num_subcores=16),
        scratch_types=(
            pltpu.VMEM((SRC_PER_W * NBR,), jnp.int32),
            pltpu.VMEM((SRC_PER_W,), jnp.int32),
            pltpu.VMEM((CHUNK_ROWS, D_NODE), jnp.float32),
            pltpu.VMEM((CHUNK_ROWS, D_NODE), jnp.float32),
            pltpu.VMEM((SRC_PER_W, D_NODE), jnp.float32),
            pltpu.VMEM((SRC_PER_W, D_NODE), jnp.float32),
            pltpu.VMEM((SRC_PER_W, D_NODE), jnp.float32),
            pltpu.SemaphoreType.DMA,
        ),
    )(_sc_nodes_body)


def _sc_edges_body(dense_hbm, eidx_hbm, out3, idx_e, gidx, gbuf, acc_t, sem):
    wid = lax.axis_index("s") * 2 + lax.axis_index("c")
    row0 = wid * (SRC_PER_W * NBR)

    pltpu.sync_copy(eidx_hbm.at[pl.ds(row0, SRC_PER_W * NBR)], idx_e)

    lanes = lax.iota(jnp.int32, 16)

    lanes20 = lanes * NBR
    zero16 = jnp.float32(0.0) + jnp.zeros_like(lanes, dtype=jnp.float32)

    def chunk_body(c, carry):
        bc = jax.lax.broadcast_in_dim(c, (16,), ()) * ECH_ROWS
        for t in range(ECH_ROWS // 16):
            cslots = lanes + 16 * t
            ev = plsc.load_gather(idx_e, [bc + cslots])
            plsc.store_scatter(gidx, [cslots], ev >> 3)
        cps = []
        for q in range(ECH_ROWS // 80):
            cps.append(pltpu.async_copy(
                dense_hbm.at[gidx.at[pl.ds(q * 80, 80)]],
                gbuf.at[pl.ds(q * 80, 80)], sem))
        for cp in cps:
            cp.wait()
        bc16 = jax.lax.broadcast_in_dim(c, (16,), ()) * ECH_SRC
        accs = [zero16 for _ in range(D_EDGE)]
        for j in range(NBR):
            rows = lanes20 + j
            sub = plsc.load_gather(idx_e, [bc + rows])
            colb = (sub & 7) * 16
            for k in range(D_EDGE):
                vals = plsc.load_gather(gbuf, [rows, colb + k])
                accs[k] = accs[k] + vals
        for k in range(D_EDGE):
            krow = jax.lax.broadcast_in_dim(jnp.int32(k), (16,), ())
            plsc.store_scatter(acc_t, [krow, bc16 + lanes], accs[k])
        return carry

    lax.fori_loop(0, ECH_N, chunk_body, 0)
    pltpu.sync_copy(acc_t, out3.at[wid])


@functools.lru_cache(maxsize=1)
def _build_sc_edges():
    return functools.partial(
        pl.kernel,
        out_type=(
            jax.ShapeDtypeStruct((NW, D_EDGE, SRC_PER_W), jnp.float32),
        ),
        mesh=plsc.VectorSubcoreMesh(core_axis_name="c", subcore_axis_name="s",
                                    num_cores=2, num_subcores=16),
        compiler_params=pltpu.CompilerParams(needs_layout_passes=False),
        scratch_types=(
            pltpu.VMEM((SRC_PER_W * NBR,), jnp.int32),
            pltpu.VMEM((ECH_ROWS,), jnp.int32),
            pltpu.VMEM((ECH_ROWS, 128), jnp.float32),
            pltpu.VMEM((D_EDGE, SRC_PER_W), jnp.float32),
            pltpu.SemaphoreType.DMA,
        ),
    )(_sc_edges_body)


def _tc_body(et_ref, nb_ref, e3_ref, sn_ref, sm_ref, w_ref, tb_ref,
             w1n_ref, w1t_ref, w1e_ref, w2n_ref, w2s_ref, w2t_ref,
             b1_ref, b2_ref, out_ref):
    ts = et_ref[:, NBR:NBR + 1]
    w = w_ref[...]
    tb = tb_ref[...]
    tsum = jnp.cos((ts - et_ref[:, 0:1]) * w + tb)
    for j in range(1, NBR):
        tsum = tsum + jnp.cos((ts - et_ref[:, j:j + 1]) * w + tb)

    dot = functools.partial(jnp.dot, preferred_element_type=jnp.float32)
    cdims = (((0,), (0,)), ((), ()))
    econ = jnp.concatenate(
        [lax.dot_general(e3_ref[u], w1e_ref[...], cdims,
                         preferred_element_type=jnp.float32)
         for u in range(2)], axis=0)
    pre = (dot(nb_ref[...], w1n_ref[...]) + dot(tsum, w1t_ref[...])
           + econ + float(NBR) * b1_ref[...])
    ns = jnp.maximum(pre, 0.0)
    src = sn_ref[...] + sm_ref[...]
    const = dot(jnp.cos(tb), w2t_ref[...]) + b2_ref[...]
    out_ref[...] = dot(ns, w2n_ref[...]) + dot(src, w2s_ref[...]) + const


def kernel(source_nodes, timestamps, neighbors, edge_idxs, edge_times,
           node_features, edge_features, memory, time_w, time_b,
           W1, b1, W2, b2):
    nbr_flat = neighbors.reshape(-1).astype(jnp.int32)
    eidx_flat = edge_idxs.reshape(-1).astype(jnp.int32)
    src = source_nodes.astype(jnp.int32)

    edge_dense = _repack_edges(edge_features)

    nb_sum, src_n, src_m = _build_sc_nodes()(
        node_features, memory, nbr_flat, src)

    e3 = _build_sc_edges()(edge_dense, eidx_flat)
    if isinstance(e3, (tuple, list)):
        e3 = e3[0]

    et_aug = jnp.concatenate([edge_times, timestamps[:, None]], axis=1)

    blk = 256
    grid = B // blk
    full = lambda i: (0, 0)
    row = lambda i: (i, 0)
    out = pl.pallas_call(
        _tc_body,
        grid=(grid,),
        in_specs=[
            pl.BlockSpec((blk, NBR + 1), row),
            pl.BlockSpec((blk, D_NODE), row),
            pl.BlockSpec((2, D_EDGE, SRC_PER_W), lambda i: (i, 0, 0)),
            pl.BlockSpec((blk, D_NODE), row),
            pl.BlockSpec((blk, D_NODE), row),
            pl.BlockSpec((1, D_TIME), full),
            pl.BlockSpec((1, D_TIME), full),
            pl.BlockSpec((D_NODE, D_EMB), full),
            pl.BlockSpec((D_TIME, D_EMB), full),
            pl.BlockSpec((D_EDGE, D_EMB), full),
            pl.BlockSpec((D_EMB, D_EMB), full),
            pl.BlockSpec((D_NODE, D_EMB), full),
            pl.BlockSpec((D_TIME, D_EMB), full),
            pl.BlockSpec((1, D_EMB), full),
            pl.BlockSpec((1, D_EMB), full),
        ],
        out_specs=pl.BlockSpec((blk, D_EMB), row),
        out_shape=jax.ShapeDtypeStruct((B, D_EMB), jnp.float32),
    )(
        et_aug, nb_sum, e3, src_n, src_m,
        time_w[None, :], time_b[None, :],
        W1[:D_NODE], W1[D_NODE:D_NODE + D_TIME], W1[D_NODE + D_TIME:],
        W2[:D_EMB], W2[D_EMB:D_EMB + D_NODE], W2[D_EMB + D_NODE:],
        b1[None, :], b2[None, :],
    )
    return out


# speedup vs baseline: 1.5529x; 1.0684x over previous
import functools

import numpy as np

import jax
import jax.numpy as jnp
from jax import lax
from jax.experimental import pallas as pl
from jax.experimental.pallas import tpu as pltpu
from jax.experimental.pallas import tpu_sc as plsc

N_NODES = 100000
N_EDGES = 1600000
D_NODE = 128
D_EDGE = 16
D_TIME = 128
D_EMB = 128
B = 4096
NBR = 20

NW = 32
SRC_PER_W = B // NW
CHUNK_SRC = 4
CHUNK_ROWS = CHUNK_SRC * NBR
N_CHUNKS = SRC_PER_W // CHUNK_SRC

EGROUPS = N_EDGES // 8
ECH_SRC = 16
ECH_ROWS = ECH_SRC * NBR
ECH_N = SRC_PER_W // ECH_SRC


RPK_CW = 3200
RPK_NC = N_EDGES // RPK_CW


def _sc_repack_body(tabT_hbm, out_hbm, bufa, bufo, sem):
    wid = lax.axis_index("s") * 2 + lax.axis_index("c")
    n_w = (RPK_NC // NW) + (wid < (RPK_NC % NW)).astype(jnp.int32)
    lanes = lax.iota(jnp.int32, 16)
    z16 = lanes * 0

    def chunk_body(i, carry):
        cid = wid + NW * i
        ioff = pl.multiple_of(cid * RPK_CW, 128)
        pltpu.sync_copy(tabT_hbm.at[:, pl.ds(ioff, RPK_CW)],
                        bufa.at[:, pl.ds(0, RPK_CW)])

        @plsc.parallel_loop(0, RPK_CW, unroll=8)
        def _x(x):
            xb = jax.lax.broadcast_in_dim(x, (16,), ())
            v = plsc.load_gather(bufa, [lanes, z16 + x])
            plsc.store_scatter(
                bufo, [xb >> 3, (xb & 7) * D_EDGE + lanes], v)
        ooff = pl.multiple_of(cid * (RPK_CW // 8), 8)
        pltpu.sync_copy(bufo, out_hbm.at[pl.ds(ooff, RPK_CW // 8)])
        return carry

    lax.fori_loop(0, n_w, chunk_body, 0)


@functools.lru_cache(maxsize=1)
def _build_sc_repack():
    return functools.partial(
        pl.kernel,
        out_type=(jax.ShapeDtypeStruct((EGROUPS, 128), jnp.float32),),
        mesh=plsc.VectorSubcoreMesh(core_axis_name="c", subcore_axis_name="s",
                                    num_cores=2, num_subcores=16),
        compiler_params=pltpu.CompilerParams(needs_layout_passes=False),
        scratch_types=(
            pltpu.VMEM((D_EDGE, RPK_CW + 1), jnp.float32),
            pltpu.VMEM((RPK_CW // 8, 128), jnp.float32),
            pltpu.SemaphoreType.DMA,
        ),
    )(_sc_repack_body)


def _repack_edges(edge_features):
    out = _build_sc_repack()(edge_features.T)
    if isinstance(out, (tuple, list)):
        out = out[0]
    return out


def _sc_nodes_body(node_hbm, mem_hbm, nbr_hbm, src_hbm,
                   out_nb, out_sn, out_sm,
                   idx_nb, idx_s, buf_n, buf_m, acc_nb, bsn, bsm, sem):
    wid = lax.axis_index("s") * 2 + lax.axis_index("c")
    row0 = wid * (SRC_PER_W * NBR)
    src0 = wid * SRC_PER_W

    pltpu.sync_copy(nbr_hbm.at[pl.ds(row0, SRC_PER_W * NBR)], idx_nb)
    pltpu.sync_copy(src_hbm.at[pl.ds(src0, SRC_PER_W)], idx_s)

    def chunk_body(c, carry):
        r = c * CHUNK_ROWS
        cp_n = pltpu.async_copy(node_hbm.at[idx_nb.at[pl.ds(r, CHUNK_ROWS)]],
                                buf_n, sem)
        cp_m = pltpu.async_copy(mem_hbm.at[idx_nb.at[pl.ds(r, CHUNK_ROWS)]],
                                buf_m, sem)
        cp_n.wait()
        cp_m.wait()
        for s in range(CHUNK_SRC):
            base = s * NBR
            orow = c * CHUNK_SRC + s
            for col in range(D_NODE // 16):
                sl = pl.ds(col * 16, 16)
                acc = buf_n[base, sl] + buf_m[base, sl]
                for j in range(1, NBR):
                    acc = acc + buf_n[base + j, sl] + buf_m[base + j, sl]
                acc_nb[orow, sl] = acc
        return carry

    lax.fori_loop(0, N_CHUNKS, chunk_body, 0)

    cp_sn = pltpu.async_copy(node_hbm.at[idx_s], bsn, sem)
    cp_sm = pltpu.async_copy(mem_hbm.at[idx_s], bsm, sem)
    cp_sn.wait()
    cp_sm.wait()

    pltpu.sync_copy(acc_nb, out_nb.at[pl.ds(src0, SRC_PER_W)])
    pltpu.sync_copy(bsn, out_sn.at[pl.ds(src0, SRC_PER_W)])
    pltpu.sync_copy(bsm, out_sm.at[pl.ds(src0, SRC_PER_W)])


@functools.lru_cache(maxsize=1)
def _build_sc_nodes():
    return functools.partial(
        pl.kernel,
        out_type=(
            jax.ShapeDtypeStruct((B, D_NODE), jnp.float32),
            jax.ShapeDtypeStruct((B, D_NODE), jnp.float32),
            jax.ShapeDtypeStruct((B, D_NODE), jnp.float32),
        ),
        mesh=plsc.VectorSubcoreMesh(core_axis_name="c", subcore_axis_name="s",
                                    num_cores=2, num_subcores=16),
        scratch_types=(
            pltpu.VMEM((SRC_PER_W * NBR,), jnp.int32),
            pltpu.VMEM((SRC_PER_W,), jnp.int32),
            pltpu.VMEM((CHUNK_ROWS, D_NODE), jnp.float32),
            pltpu.VMEM((CHUNK_ROWS, D_NODE), jnp.float32),
            pltpu.VMEM((SRC_PER_W, D_NODE), jnp.float32),
            pltpu.VMEM((SRC_PER_W, D_NODE), jnp.float32),
            pltpu.VMEM((SRC_PER_W, D_NODE), jnp.float32),
            pltpu.SemaphoreType.DMA,
        ),
    )(_sc_nodes_body)


def _sc_edges_body(dense_hbm, eidx_hbm, out3, idx_e, gidx, gbuf, acc_t, sem):
    wid = lax.axis_index("s") * 2 + lax.axis_index("c")
    row0 = wid * (SRC_PER_W * NBR)

    pltpu.sync_copy(eidx_hbm.at[pl.ds(row0, SRC_PER_W * NBR)], idx_e)

    lanes = lax.iota(jnp.int32, 16)

    lanes20 = lanes * NBR
    zero16 = jnp.float32(0.0) + jnp.zeros_like(lanes, dtype=jnp.float32)

    def chunk_body(c, carry):
        bc = jax.lax.broadcast_in_dim(c, (16,), ()) * ECH_ROWS
        for t in range(ECH_ROWS // 16):
            cslots = lanes + 16 * t
            ev = plsc.load_gather(idx_e, [bc + cslots])
            plsc.store_scatter(gidx, [cslots], ev >> 3)
        cps = []
        for q in range(ECH_ROWS // 80):
            cps.append(pltpu.async_copy(
                dense_hbm.at[gidx.at[pl.ds(q * 80, 80)]],
                gbuf.at[pl.ds(q * 80, 80)], sem))
        for cp in cps:
            cp.wait()
        bc16 = jax.lax.broadcast_in_dim(c, (16,), ()) * ECH_SRC
        accs = [zero16 for _ in range(D_EDGE)]
        for j in range(NBR):
            rows = lanes20 + j
            sub = plsc.load_gather(idx_e, [bc + rows])
            colb = (sub & 7) * 16
            for k in range(D_EDGE):
                vals = plsc.load_gather(gbuf, [rows, colb + k])
                accs[k] = accs[k] + vals
        for k in range(D_EDGE):
            krow = jax.lax.broadcast_in_dim(jnp.int32(k), (16,), ())
            plsc.store_scatter(acc_t, [krow, bc16 + lanes], accs[k])
        return carry

    lax.fori_loop(0, ECH_N, chunk_body, 0)
    pltpu.sync_copy(acc_t, out3.at[wid])


@functools.lru_cache(maxsize=1)
def _build_sc_edges():
    return functools.partial(
        pl.kernel,
        out_type=(
            jax.ShapeDtypeStruct((NW, D_EDGE, SRC_PER_W), jnp.float32),
        ),
        mesh=plsc.VectorSubcoreMesh(core_axis_name="c", subcore_axis_name="s",
                                    num_cores=2, num_subcores=16),
        compiler_params=pltpu.CompilerParams(needs_layout_passes=False),
        scratch_types=(
            pltpu.VMEM((SRC_PER_W * NBR,), jnp.int32),
            pltpu.VMEM((ECH_ROWS,), jnp.int32),
            pltpu.VMEM((ECH_ROWS, 128), jnp.float32),
            pltpu.VMEM((D_EDGE, SRC_PER_W), jnp.float32),
            pltpu.SemaphoreType.DMA,
        ),
    )(_sc_edges_body)


def _tc_body(et_ref, nb_ref, e3_ref, sn_ref, sm_ref, w_ref, tb_ref,
             w1n_ref, w1t_ref, w1e_ref, w2n_ref, w2s_ref, w2t_ref,
             b1_ref, b2_ref, out_ref):
    ts = et_ref[:, NBR:NBR + 1]
    w = w_ref[...]
    tb = tb_ref[...]
    tsum = jnp.cos((ts - et_ref[:, 0:1]) * w + tb)
    for j in range(1, NBR):
        tsum = tsum + jnp.cos((ts - et_ref[:, j:j + 1]) * w + tb)

    dot = functools.partial(jnp.dot, preferred_element_type=jnp.float32)
    cdims = (((0,), (0,)), ((), ()))
    econ = jnp.concatenate(
        [lax.dot_general(e3_ref[u], w1e_ref[...], cdims,
                         preferred_element_type=jnp.float32)
         for u in range(2)], axis=0)
    pre = (dot(nb_ref[...], w1n_ref[...]) + dot(tsum, w1t_ref[...])
           + econ + float(NBR) * b1_ref[...])
    ns = jnp.maximum(pre, 0.0)
    src = sn_ref[...] + sm_ref[...]
    const = dot(jnp.cos(tb), w2t_ref[...]) + b2_ref[...]
    out_ref[...] = dot(ns, w2n_ref[...]) + dot(src, w2s_ref[...]) + const


def kernel(source_nodes, timestamps, neighbors, edge_idxs, edge_times,
           node_features, edge_features, memory, time_w, time_b,
           W1, b1, W2, b2):
    nbr_flat = neighbors.reshape(-1).astype(jnp.int32)
    eidx_flat = edge_idxs.reshape(-1).astype(jnp.int32)
    src = source_nodes.astype(jnp.int32)

    edge_dense = _repack_edges(edge_features)

    nb_sum, src_n, src_m = _build_sc_nodes()(
        node_features, memory, nbr_flat, src)

    e3 = _build_sc_edges()(edge_dense, eidx_flat)
    if isinstance(e3, (tuple, list)):
        e3 = e3[0]

    et_aug = jnp.concatenate([edge_times, timestamps[:, None]], axis=1)

    blk = 256
    grid = B // blk
    full = lambda i: (0, 0)
    row = lambda i: (i, 0)
    out = pl.pallas_call(
        _tc_body,
        grid=(grid,),
        in_specs=[
            pl.BlockSpec((blk, NBR + 1), row),
            pl.BlockSpec((blk, D_NODE), row),
            pl.BlockSpec((2, D_EDGE, SRC_PER_W), lambda i: (i, 0, 0)),
            pl.BlockSpec((blk, D_NODE), row),
            pl.BlockSpec((blk, D_NODE), row),
            pl.BlockSpec((1, D_TIME), full),
            pl.BlockSpec((1, D_TIME), full),
            pl.BlockSpec((D_NODE, D_EMB), full),
            pl.BlockSpec((D_TIME, D_EMB), full),
            pl.BlockSpec((D_EDGE, D_EMB), full),
            pl.BlockSpec((D_EMB, D_EMB), full),
            pl.BlockSpec((D_NODE, D_EMB), full),
            pl.BlockSpec((D_TIME, D_EMB), full),
            pl.BlockSpec((1, D_EMB), full),
            pl.BlockSpec((1, D_EMB), full),
        ],
        out_specs=pl.BlockSpec((blk, D_EMB), row),
        out_shape=jax.ShapeDtypeStruct((B, D_EMB), jnp.float32),
    )(
        et_aug, nb_sum, e3, src_n, src_m,
        time_w[None, :], time_b[None, :],
        W1[:D_NODE], W1[D_NODE:D_NODE + D_TIME], W1[D_NODE + D_TIME:],
        W2[:D_EMB], W2[D_EMB:D_EMB + D_NODE], W2[D_EMB + D_NODE:],
        b1[None, :], b2[None, :],
    )
    return out


# speedup vs baseline: 1.7590x; 1.1327x over previous
import functools

import numpy as np

import jax
import jax.numpy as jnp
from jax import lax
from jax.experimental import pallas as pl
from jax.experimental.pallas import tpu as pltpu
from jax.experimental.pallas import tpu_sc as plsc

N_NODES = 100000
N_EDGES = 1600000
D_NODE = 128
D_EDGE = 16
D_TIME = 128
D_EMB = 128
B = 4096
NBR = 20

NW = 32
SRC_PER_W = B // NW
CHUNK_SRC = 4
CHUNK_ROWS = CHUNK_SRC * NBR
N_CHUNKS = SRC_PER_W // CHUNK_SRC

EGROUPS = N_EDGES // 8
ECH_SRC = 16
ECH_ROWS = ECH_SRC * NBR
ECH_N = SRC_PER_W // ECH_SRC


RPK_CW = 3200
RPK_NC = N_EDGES // RPK_CW


def _sc_repack_body(tabT_hbm, out_hbm, bufa, bufo, sem):
    wid = lax.axis_index("s") * 2 + lax.axis_index("c")
    n_w = (RPK_NC // NW) + (wid < (RPK_NC % NW)).astype(jnp.int32)
    lanes = lax.iota(jnp.int32, 16)
    z16 = lanes * 0

    def chunk_body(i, carry):
        cid = wid + NW * i
        ioff = pl.multiple_of(cid * RPK_CW, 128)
        pltpu.sync_copy(tabT_hbm.at[:, pl.ds(ioff, RPK_CW)],
                        bufa.at[:, pl.ds(0, RPK_CW)])

        @plsc.parallel_loop(0, RPK_CW, unroll=8)
        def _x(x):
            xb = jax.lax.broadcast_in_dim(x, (16,), ())
            v = plsc.load_gather(bufa, [lanes, z16 + x])
            plsc.store_scatter(
                bufo, [xb >> 3, (xb & 7) * D_EDGE + lanes], v)
        ooff = pl.multiple_of(cid * (RPK_CW // 8), 8)
        pltpu.sync_copy(bufo, out_hbm.at[pl.ds(ooff, RPK_CW // 8)])
        return carry

    lax.fori_loop(0, n_w, chunk_body, 0)


@functools.lru_cache(maxsize=1)
def _build_sc_repack():
    return functools.partial(
        pl.kernel,
        out_type=(jax.ShapeDtypeStruct((EGROUPS, 128), jnp.float32),),
        mesh=plsc.VectorSubcoreMesh(core_axis_name="c", subcore_axis_name="s",
                                    num_cores=2, num_subcores=16),
        compiler_params=pltpu.CompilerParams(needs_layout_passes=False),
        scratch_types=(
            pltpu.VMEM((D_EDGE, RPK_CW + 1), jnp.float32),
            pltpu.VMEM((RPK_CW // 8, 128), jnp.float32),
            pltpu.SemaphoreType.DMA,
        ),
    )(_sc_repack_body)


def _repack_edges(edge_features):
    out = _build_sc_repack()(edge_features.T)
    if isinstance(out, (tuple, list)):
        out = out[0]
    return out


def _sc_nodes_body(node_hbm, mem_hbm, nbr_hbm, src_hbm,
                   out_nb, out_sn, out_sm,
                   idx_nb, idx_s, buf_n, buf_m, acc_nb, bsn, bsm, sem):
    wid = lax.axis_index("s") * 2 + lax.axis_index("c")
    row0 = wid * (SRC_PER_W * NBR)
    src0 = wid * SRC_PER_W

    pltpu.sync_copy(nbr_hbm.at[pl.ds(row0, SRC_PER_W * NBR)], idx_nb)
    pltpu.sync_copy(src_hbm.at[pl.ds(src0, SRC_PER_W)], idx_s)

    def chunk_body(c, carry):
        r = c * CHUNK_ROWS
        cp_n = pltpu.async_copy(node_hbm.at[idx_nb.at[pl.ds(r, CHUNK_ROWS)]],
                                buf_n, sem)
        cp_m = pltpu.async_copy(mem_hbm.at[idx_nb.at[pl.ds(r, CHUNK_ROWS)]],
                                buf_m, sem)
        cp_n.wait()
        cp_m.wait()
        for s in range(CHUNK_SRC):
            base = s * NBR
            orow = c * CHUNK_SRC + s
            for col in range(D_NODE // 16):
                sl = pl.ds(col * 16, 16)
                acc = buf_n[base, sl] + buf_m[base, sl]
                for j in range(1, NBR):
                    acc = acc + buf_n[base + j, sl] + buf_m[base + j, sl]
                acc_nb[orow, sl] = acc
        return carry

    lax.fori_loop(0, N_CHUNKS, chunk_body, 0)

    cp_sn = pltpu.async_copy(node_hbm.at[idx_s], bsn, sem)
    cp_sm = pltpu.async_copy(mem_hbm.at[idx_s], bsm, sem)
    cp_sn.wait()
    cp_sm.wait()

    pltpu.sync_copy(acc_nb, out_nb.at[pl.ds(src0, SRC_PER_W)])
    pltpu.sync_copy(bsn, out_sn.at[pl.ds(src0, SRC_PER_W)])
    pltpu.sync_copy(bsm, out_sm.at[pl.ds(src0, SRC_PER_W)])


@functools.lru_cache(maxsize=1)
def _build_sc_nodes():
    return functools.partial(
        pl.kernel,
        out_type=(
            jax.ShapeDtypeStruct((B, D_NODE), jnp.float32),
            jax.ShapeDtypeStruct((B, D_NODE), jnp.float32),
            jax.ShapeDtypeStruct((B, D_NODE), jnp.float32),
        ),
        mesh=plsc.VectorSubcoreMesh(core_axis_name="c", subcore_axis_name="s",
                                    num_cores=2, num_subcores=16),
        scratch_types=(
            pltpu.VMEM((SRC_PER_W * NBR,), jnp.int32),
            pltpu.VMEM((SRC_PER_W,), jnp.int32),
            pltpu.VMEM((CHUNK_ROWS, D_NODE), jnp.float32),
            pltpu.VMEM((CHUNK_ROWS, D_NODE), jnp.float32),
            pltpu.VMEM((SRC_PER_W, D_NODE), jnp.float32),
            pltpu.VMEM((SRC_PER_W, D_NODE), jnp.float32),
            pltpu.VMEM((SRC_PER_W, D_NODE), jnp.float32),
            pltpu.SemaphoreType.DMA,
        ),
    )(_sc_nodes_body)


def _sc_edges_body(dense_hbm, eidx_hbm, out3, idx_e, gidx, gbuf, acc_t, sem):
    wid = lax.axis_index("s") * 2 + lax.axis_index("c")
    row0 = wid * (SRC_PER_W * NBR)

    pltpu.sync_copy(eidx_hbm.at[pl.ds(row0, SRC_PER_W * NBR)], idx_e)

    lanes = lax.iota(jnp.int32, 16)

    lanes20 = lanes * NBR
    zero16 = jnp.float32(0.0) + jnp.zeros_like(lanes, dtype=jnp.float32)

    def chunk_body(c, carry):
        bc = jax.lax.broadcast_in_dim(c, (16,), ()) * ECH_ROWS
        for t in range(ECH_ROWS // 16):
            cslots = lanes + 16 * t
            ev = plsc.load_gather(idx_e, [bc + cslots])
            plsc.store_scatter(gidx, [cslots], ev >> 3)
        cps = []
        for q in range(ECH_ROWS // 80):
            cps.append(pltpu.async_copy(
                dense_hbm.at[gidx.at[pl.ds(q * 80, 80)]],
                gbuf.at[pl.ds(q * 80, 80)], sem))
        for cp in cps:
            cp.wait()
        bc16 = jax.lax.broadcast_in_dim(c, (16,), ()) * ECH_SRC
        accs = [zero16 for _ in range(D_EDGE)]
        for j in range(NBR):
            rows = lanes20 + j
            sub = plsc.load_gather(idx_e, [bc + rows])
            colb = (sub & 7) * 16
            for k in range(D_EDGE):
                vals = plsc.load_gather(gbuf, [rows, colb + k])
                accs[k] = accs[k] + vals
        for k in range(D_EDGE):
            krow = jax.lax.broadcast_in_dim(jnp.int32(k), (16,), ())
            plsc.store_scatter(acc_t, [krow, bc16 + lanes], accs[k])
        return carry

    lax.fori_loop(0, ECH_N, chunk_body, 0)
    pltpu.sync_copy(acc_t, out3.at[wid])


@functools.lru_cache(maxsize=1)
def _build_sc_edges():
    return functools.partial(
        pl.kernel,
        out_type=(
            jax.ShapeDtypeStruct((NW, D_EDGE, SRC_PER_W), jnp.float32),
        ),
        mesh=plsc.VectorSubcoreMesh(core_axis_name="c", subcore_axis_name="s",
                                    num_cores=2, num_subcores=16),
        compiler_params=pltpu.CompilerParams(needs_layout_passes=False),
        scratch_types=(
            pltpu.VMEM((SRC_PER_W * NBR,), jnp.int32),
            pltpu.VMEM((ECH_ROWS,), jnp.int32),
            pltpu.VMEM((ECH_ROWS, 128), jnp.float32),
            pltpu.VMEM((D_EDGE, SRC_PER_W), jnp.float32),
            pltpu.SemaphoreType.DMA,
        ),
    )(_sc_edges_body)


_COS_C = (0.999999987933947, -19.739204194876663, 64.93910427488322,
          -85.44993350539751, 60.16619609794873, -25.96306708225088,
          6.523394091882683)
_MAGIC = float(1.5 * 2 ** 23)


def _cos2pi(u):
    r = (u + _MAGIC) - _MAGIC
    v = u - r
    s = v * v
    p = jnp.float32(_COS_C[6])
    for c in _COS_C[5::-1]:
        p = p * s + jnp.float32(c)
    return p


def _tc_body(et_ref, nb_ref, e3_ref, sn_ref, sm_ref, w2_ref, tb2_ref, tb_ref,
             w1n_ref, w1t_ref, w1e_ref, w2n_ref, w2s_ref, w2t_ref,
             b1_ref, b2_ref, out_ref):
    ts = et_ref[:, NBR:NBR + 1]
    w2 = w2_ref[...]
    tb2 = tb2_ref[...]
    tb = tb_ref[...]
    tsum = _cos2pi((ts - et_ref[:, 0:1]) * w2 + tb2)
    for j in range(1, NBR):
        tsum = tsum + _cos2pi((ts - et_ref[:, j:j + 1]) * w2 + tb2)

    dot = functools.partial(jnp.dot, preferred_element_type=jnp.float32)
    cdims = (((0,), (0,)), ((), ()))
    econ = jnp.concatenate(
        [lax.dot_general(e3_ref[u], w1e_ref[...], cdims,
                         preferred_element_type=jnp.float32)
         for u in range(2)], axis=0)
    pre = (dot(nb_ref[...], w1n_ref[...]) + dot(tsum, w1t_ref[...])
           + econ + float(NBR) * b1_ref[...])
    ns = jnp.maximum(pre, 0.0)
    src = sn_ref[...] + sm_ref[...]
    const = dot(jnp.cos(tb), w2t_ref[...]) + b2_ref[...]
    out_ref[...] = dot(ns, w2n_ref[...]) + dot(src, w2s_ref[...]) + const


def kernel(source_nodes, timestamps, neighbors, edge_idxs, edge_times,
           node_features, edge_features, memory, time_w, time_b,
           W1, b1, W2, b2):
    nbr_flat = neighbors.reshape(-1).astype(jnp.int32)
    eidx_flat = edge_idxs.reshape(-1).astype(jnp.int32)
    src = source_nodes.astype(jnp.int32)

    edge_dense = _repack_edges(edge_features)

    nb_sum, src_n, src_m = _build_sc_nodes()(
        node_features, memory, nbr_flat, src)

    e3 = _build_sc_edges()(edge_dense, eidx_flat)
    if isinstance(e3, (tuple, list)):
        e3 = e3[0]

    et_aug = jnp.concatenate([edge_times, timestamps[:, None]], axis=1)

    blk = 256
    grid = B // blk
    full = lambda i: (0, 0)
    row = lambda i: (i, 0)
    out = pl.pallas_call(
        _tc_body,
        grid=(grid,),
        in_specs=[
            pl.BlockSpec((blk, NBR + 1), row),
            pl.BlockSpec((blk, D_NODE), row),
            pl.BlockSpec((2, D_EDGE, SRC_PER_W), lambda i: (i, 0, 0)),
            pl.BlockSpec((blk, D_NODE), row),
            pl.BlockSpec((blk, D_NODE), row),
            pl.BlockSpec((1, D_TIME), full),
            pl.BlockSpec((1, D_TIME), full),
            pl.BlockSpec((1, D_TIME), full),
            pl.BlockSpec((D_NODE, D_EMB), full),
            pl.BlockSpec((D_TIME, D_EMB), full),
            pl.BlockSpec((D_EDGE, D_EMB), full),
            pl.BlockSpec((D_EMB, D_EMB), full),
            pl.BlockSpec((D_NODE, D_EMB), full),
            pl.BlockSpec((D_TIME, D_EMB), full),
            pl.BlockSpec((1, D_EMB), full),
            pl.BlockSpec((1, D_EMB), full),
        ],
        out_specs=pl.BlockSpec((blk, D_EMB), row),
        out_shape=jax.ShapeDtypeStruct((B, D_EMB), jnp.float32),
    )(
        et_aug, nb_sum, e3, src_n, src_m,
        (time_w / (2.0 * np.pi))[None, :], (time_b / (2.0 * np.pi))[None, :],
        time_b[None, :],
        W1[:D_NODE], W1[D_NODE:D_NODE + D_TIME], W1[D_NODE + D_TIME:],
        W2[:D_EMB], W2[D_EMB:D_EMB + D_NODE], W2[D_EMB + D_NODE:],
        b1[None, :], b2[None, :],
    )
    return out
